# baseline (speedup 1.0000x reference)
import math

import jax
import jax.numpy as jnp
from jax import lax
from jax.experimental import pallas as pl
from jax.experimental.pallas import tpu as pltpu
from jax.experimental.pallas import tpu_sc as plsc

D_MODEL = 128
SCALE = math.sqrt(D_MODEL)
NUM_CORES = 2
NUM_SUBCORES = 16
NUM_WORKERS = NUM_CORES * NUM_SUBCORES
CHUNK = 128
LANES = 16


NBUF = 5


def _emb_body(x_hbm, table_hbm, out_hbm, idx_v, bufs, *sems):
    wid = lax.axis_index("s") * NUM_CORES + lax.axis_index("c")
    num_chunks = idx_v.shape[0]
    gsem = sems[:NBUF]
    psem = sems[NBUF:]
    pltpu.sync_copy(x_hbm.at[wid], idx_v)

    def issue_gather(s, b):
        pltpu.async_copy(table_hbm.at[idx_v.at[s]], bufs.at[b], gsem[b])

    def drain_put(b):
        pltpu.make_async_copy(bufs.at[b], out_slice(0), psem[b]).wait()

    def out_slice(s):
        return out_hbm.at[pl.ds((wid * num_chunks + s) * CHUNK, CHUNK)]

    for b in range(3):
        issue_gather(b, b)

    def do_group(g, carry):
        for b in range(NBUF):
            s = NBUF * g + b
            nxt = (b + 3) % NBUF

            @pl.when(s >= 2)
            def _():
                drain_put(nxt)

            @pl.when(s + 3 < num_chunks)
            def _():
                issue_gather(s + 3, nxt)

            pltpu.make_async_copy(table_hbm.at[idx_v.at[s]], bufs.at[b],
                                  gsem[b]).wait()

            def scale_row(r, c2):
                for j in range(D_MODEL // LANES):
                    sl = pl.ds(j * LANES, LANES)
                    bufs[b, r, sl] = bufs[b, r, sl] * SCALE
                return c2

            lax.fori_loop(0, CHUNK, scale_row, 0, unroll=2)
            pltpu.async_copy(bufs.at[b], out_slice(s), psem[b])
        return carry

    lax.fori_loop(0, num_chunks // NBUF, do_group, 0)
    drain_put((num_chunks - 2) % NBUF)
    drain_put((num_chunks - 1) % NBUF)


def kernel(x, table):
    batch, hist = x.shape
    vocab, d = table.shape
    total = batch * hist
    assert d == D_MODEL and total % (NUM_WORKERS * CHUNK * NBUF) == 0
    s_chunks = total // (NUM_WORKERS * CHUNK)

    xt = x.T.reshape(NUM_WORKERS, s_chunks, CHUNK).astype(jnp.int32)
    mesh = plsc.VectorSubcoreMesh(core_axis_name="c", subcore_axis_name="s")
    flat = pl.kernel(
        _emb_body,
        out_type=jax.ShapeDtypeStruct((total, D_MODEL), jnp.float32),
        mesh=mesh,
        scratch_types=[
            pltpu.VMEM((s_chunks, CHUNK), jnp.int32),
            pltpu.VMEM((NBUF, CHUNK, D_MODEL), jnp.float32),
        ] + [pltpu.SemaphoreType.DMA] * (2 * NBUF),
    )(xt, table)
    return flat.reshape(hist, batch, D_MODEL).swapaxes(0, 1)
